# Initial kernel scaffold; baseline (speedup 1.0000x reference)
import jax
import jax.numpy as jnp
from jax.experimental import pallas as pl


def kernel(x, edge_index, emb_tables, conv1_W, conv1_b, conv2_W, conv2_b, head_W, head_b):
    raise NotImplementedError("write your pallas kernel here")


# speedup vs baseline: 37.0791x; 37.0791x over previous
import functools

import jax
import jax.numpy as jnp
from jax import lax
from jax.experimental import pallas as pl
from jax.experimental.pallas import tpu as pltpu
from jax.experimental.pallas import tpu_sc as plsc

N_TYPES = 100
BATCH = 1024
TOTAL = N_TYPES * BATCH
E = TOTAL * 16
EMB = 16
HID = 32
NCLS = 10
NC, NS, L = 2, 16, 16
NW = NC * NS
ER = E // 128
F32 = jnp.float32
I32 = jnp.int32

_MESH = functools.partial(
    plsc.VectorSubcoreMesh, core_axis_name="c", subcore_axis_name="s",
    num_cores=NC, num_subcores=NS)
_SC_PARAMS = pltpu.CompilerParams(use_tc_tiling_on_sc=False)


def _hds(sds):
    return jax.ShapeDtypeStruct(sds[0], sds[1])


def _edge_tx_body(e_ref, o_ref):
    i = e_ref[...]
    q0 = (i.astype(F32) * 0.01).astype(I32)
    r0 = i - q0 * 100
    ge = (r0 >= 100).astype(I32)
    lt = (r0 < 0).astype(I32)
    q = q0 + ge - lt
    r = r0 - 100 * ge + 100 * lt
    o_ref[...] = r * BATCH + q


def _edge_transform(edge_index):
    BE = 65536
    return pl.pallas_call(
        _edge_tx_body,
        grid=(E // BE,),
        in_specs=[pl.BlockSpec((2, BE), lambda g: (0, g))],
        out_specs=pl.BlockSpec((2, BE), lambda g: (0, g)),
        out_shape=jax.ShapeDtypeStruct((2, E), I32),
    )(edge_index)


def _deg_emb_body(dst2, xt, embt, degp, h0, dacc, zb, ones128, dbuf, xbuf,
                  fidx, rows, gsem):
    c = lax.axis_index("c")
    s = lax.axis_index("s")
    wid = s * NC + c

    def zz(i, _):
        zb[pl.ds(i * 16, 16)] = jnp.zeros((16,), F32)
        return _
    lax.fori_loop(0, 50, zz, None)
    for k in range(8):
        pltpu.sync_copy(zb, dacc.at[pl.ds(s * 6400 + k * 800, 800)])

    def oo(i, _):
        ones128[pl.ds(i * 16, 16)] = jnp.ones((16,), F32)
        return _
    lax.fori_loop(0, 8, oo, None)
    plsc.subcore_barrier()

    rows_per_tile = ER // NW
    def dchunk(t, _):
        rb = wid * rows_per_tile + t * 8
        pltpu.sync_copy(dst2.at[pl.ds(rb, 8)], dbuf)
        for j in range(8):
            pltpu.sync_copy(ones128, dacc.at[dbuf.at[j]], add=True)
        return _
    lax.fori_loop(0, rows_per_tile // 8, dchunk, None)

    iota = lax.iota(I32, 16)
    nb0 = wid * (TOTAL // NW)
    def echunk(ch, _):
        nb = nb0 + ch * 640
        pltpu.sync_copy(xt.at[pl.ds(nb, 640)], xbuf)
        for g in range(40):
            xv = xbuf[pl.ds(g * 16, 16)]
            xv = jnp.minimum(jnp.maximum(xv, 0), NCLS)
            row = nb + g * 16 + iota
            fi = (row >> 10) * (NCLS + 1) + xv
            fidx[g // 8, pl.ds((g % 8) * 16, 16)] = fi
        for rr in range(5):
            pltpu.async_copy(embt.at[fidx.at[rr]],
                             rows.at[pl.ds(rr * 128, 128)], gsem).wait()
        pltpu.sync_copy(rows, h0.at[pl.ds(nb, 640)])
        return _
    lax.fori_loop(0, 5, echunk, None)

    plsc.subcore_barrier()
    off = c * TOTAL + s * 6400
    pltpu.sync_copy(dacc.at[pl.ds(s * 6400, 6400)], degp.at[pl.ds(off, 6400)])


def _deg_emb(dst2, xt, embt):
    return pl.kernel(
        _deg_emb_body,
        out_type=(jax.ShapeDtypeStruct((NC * TOTAL,), F32),
                  jax.ShapeDtypeStruct((TOTAL, EMB), F32)),
        mesh=_MESH(),
        compiler_params=_SC_PARAMS,
        scratch_types=(
            pltpu.VMEM_SHARED((TOTAL,), F32),
            pltpu.VMEM((800,), F32),
            pltpu.VMEM((128,), F32),
            pltpu.VMEM((8, 128), I32),
            pltpu.VMEM((640,), I32),
            pltpu.VMEM((5, 128), I32),
            pltpu.VMEM((640, EMB), F32),
            pltpu.SemaphoreType.DMA,
        ),
    )(dst2, xt, embt)


def _agg_body(per_sc_all_edges, src2, dst2, ta, tb, parts, sacc, zb, sbuf,
              dbuf, grow0, grow1, gsem0, gsem1):
    c = lax.axis_index("c")
    s = lax.axis_index("s")
    wid = s * NC + c

    def zz(i, _):
        zb[i] = jnp.zeros((16,), F32)
        return _
    lax.fori_loop(0, 400, zz, None)
    for k in range(16):
        pltpu.sync_copy(zb, sacc.at[pl.ds(s * 6400 + k * 400, 400)])
    plsc.subcore_barrier()

    def edge_loop(table, row_base, nrows):
        grows = (grow0, grow1)
        gsems = (gsem0, gsem1)
        def chunk(t, _):
            rb = row_base + t * 8
            pltpu.sync_copy(src2.at[pl.ds(rb, 8)], sbuf)
            pltpu.sync_copy(dst2.at[pl.ds(rb, 8)], dbuf)
            cp = pltpu.async_copy(table.at[sbuf.at[0]], grows[0], gsems[0])
            for j in range(8):
                if j < 7:
                    nxt = pltpu.async_copy(table.at[sbuf.at[j + 1]],
                                           grows[(j + 1) % 2],
                                           gsems[(j + 1) % 2])
                cp.wait()
                pltpu.sync_copy(grows[j % 2], sacc.at[dbuf.at[j]], add=True)
                if j < 7:
                    cp = nxt
            return _
        lax.fori_loop(0, nrows // 8, chunk, None)

    if per_sc_all_edges:
        @pl.when(c == 0)
        def _():
            edge_loop(ta, s * (ER // NS), ER // NS)
        @pl.when(c == 1)
        def _():
            edge_loop(tb, s * (ER // NS), ER // NS)
    else:
        edge_loop(ta, wid * (ER // NW), ER // NW)

    plsc.subcore_barrier()
    off = c * TOTAL + s * 6400
    pltpu.sync_copy(sacc.at[pl.ds(s * 6400, 6400)],
                    parts.at[pl.ds(off, 6400)])


def _agg(src2, dst2, ta, tb, per_sc_all_edges):
    return pl.kernel(
        functools.partial(_agg_body, per_sc_all_edges),
        out_type=jax.ShapeDtypeStruct((NC * TOTAL, EMB), F32),
        mesh=_MESH(),
        compiler_params=_SC_PARAMS,
        scratch_types=(
            pltpu.VMEM_SHARED((TOTAL, EMB), F32),
            pltpu.VMEM((400, EMB), F32),
            pltpu.VMEM((8, 128), I32),
            pltpu.VMEM((8, 128), I32),
            pltpu.VMEM((128, EMB), F32),
            pltpu.VMEM((128, EMB), F32),
            pltpu.SemaphoreType.DMA,
            pltpu.SemaphoreType.DMA,
        ),
    )(src2, dst2, ta, tb)


def _dinv_body(degp_ref, h0_ref, dinv_ref, g0_ref):
    deg = degp_ref[0] + degp_ref[1] + 1.0
    dinv = lax.rsqrt(deg)[:, None]
    dinv_ref[...] = dinv
    g0_ref[...] = dinv * h0_ref[...]


def _dinv_g0(degp, h0):
    BK = 4096
    return pl.pallas_call(
        _dinv_body,
        grid=(TOTAL // BK,),
        in_specs=[pl.BlockSpec((2, BK), lambda g: (0, g)),
                  pl.BlockSpec((BK, EMB), lambda g: (g, 0))],
        out_specs=[pl.BlockSpec((BK, 1), lambda g: (g, 0)),
                   pl.BlockSpec((BK, EMB), lambda g: (g, 0))],
        out_shape=(jax.ShapeDtypeStruct((TOTAL, 1), F32),
                   jax.ShapeDtypeStruct((TOTAL, EMB), F32)),
    )(degp.reshape(2, TOTAL), h0)


def _layer1_body(parts_ref, g0_ref, dinv_ref, w1t_ref, b1_ref,
                 g1a_ref, g1b_ref):
    ssum = parts_ref[0] + parts_ref[1] + g0_ref[...]
    agg = dinv_ref[...] * ssum
    h1 = jnp.dot(agg, w1t_ref[...], precision=lax.Precision.HIGHEST,
                 preferred_element_type=F32) + b1_ref[...]
    h1 = jnp.maximum(h1, 0.0)
    g1 = dinv_ref[...] * h1
    g1a_ref[...] = g1[:, :EMB]
    g1b_ref[...] = g1[:, EMB:]


def _layer1(parts, g0, dinv, w1t, b1):
    BK = 4096
    return pl.pallas_call(
        _layer1_body,
        grid=(TOTAL // BK,),
        in_specs=[pl.BlockSpec((2, BK, EMB), lambda g: (0, g, 0)),
                  pl.BlockSpec((BK, EMB), lambda g: (g, 0)),
                  pl.BlockSpec((BK, 1), lambda g: (g, 0)),
                  pl.BlockSpec((EMB, HID), lambda g: (0, 0)),
                  pl.BlockSpec((1, HID), lambda g: (0, 0))],
        out_specs=[pl.BlockSpec((BK, EMB), lambda g: (g, 0)),
                   pl.BlockSpec((BK, EMB), lambda g: (g, 0))],
        out_shape=(jax.ShapeDtypeStruct((TOTAL, EMB), F32),
                   jax.ShapeDtypeStruct((TOTAL, EMB), F32)),
    )(parts.reshape(2, TOTAL, EMB), g0, dinv, w1t, b1)


def _layer2_body(parts_ref, g1a_ref, g1b_ref, dinv_ref, w2t_ref, b2_ref,
                 hwt_ref, hb_ref, out_ref):
    h0c = parts_ref[0] + g1a_ref[...]
    h1c = parts_ref[1] + g1b_ref[...]
    agg = dinv_ref[...] * jnp.concatenate([h0c, h1c], axis=1)
    h2 = jnp.dot(agg, w2t_ref[...], precision=lax.Precision.HIGHEST,
                 preferred_element_type=F32) + b2_ref[...]
    lg = jnp.dot(h2, hwt_ref[0], precision=lax.Precision.HIGHEST,
                 preferred_element_type=F32) + hb_ref[0]
    out_ref[0] = lg


def _layer2_head(parts, g1a, g1b, dinv, w2t, b2, hwt, hb):
    return pl.pallas_call(
        _layer2_body,
        grid=(N_TYPES,),
        in_specs=[pl.BlockSpec((2, BATCH, EMB), lambda n: (0, n, 0)),
                  pl.BlockSpec((BATCH, EMB), lambda n: (n, 0)),
                  pl.BlockSpec((BATCH, EMB), lambda n: (n, 0)),
                  pl.BlockSpec((BATCH, 1), lambda n: (n, 0)),
                  pl.BlockSpec((HID, HID), lambda n: (0, 0)),
                  pl.BlockSpec((1, HID), lambda n: (0, 0)),
                  pl.BlockSpec((1, HID, NCLS), lambda n: (n, 0, 0)),
                  pl.BlockSpec((1, 1, NCLS), lambda n: (n, 0, 0))],
        out_specs=pl.BlockSpec((1, BATCH, NCLS), lambda n: (n, 0, 0)),
        out_shape=jax.ShapeDtypeStruct((N_TYPES, BATCH, NCLS), F32),
    )(parts.reshape(2, TOTAL, EMB), g1a, g1b, dinv, w2t, b2, hwt,
      hb.reshape(N_TYPES, 1, NCLS))


def kernel(x, edge_index, emb_tables, conv1_W, conv1_b, conv2_W, conv2_b,
           head_W, head_b):
    xt = x.reshape(BATCH, N_TYPES).T.reshape(TOTAL)
    embt = emb_tables.reshape(N_TYPES * (NCLS + 1), EMB)
    w1t = conv1_W.T
    b1 = conv1_b.reshape(1, HID)
    w2t = conv2_W.T
    b2 = conv2_b.reshape(1, HID)
    hwt = head_W.transpose(0, 2, 1)

    et = _edge_transform(edge_index)
    src2 = et[0].reshape(ER, 128)
    dst2 = et[1].reshape(ER, 128)

    degp, h0 = _deg_emb(dst2, xt, embt)
    dinv, g0 = _dinv_g0(degp, h0)
    parts1 = _agg(src2, dst2, g0, g0, per_sc_all_edges=False)
    g1a, g1b = _layer1(parts1, g0, dinv, w1t, b1)
    parts2 = _agg(src2, dst2, g1a, g1b, per_sc_all_edges=True)
    lg = _layer2_head(parts2, g1a, g1b, dinv, w2t, b2, hwt, head_b)
    return lg.transpose(1, 0, 2)


# speedup vs baseline: 41.4966x; 1.1191x over previous
import functools

import jax
import jax.numpy as jnp
from jax import lax
from jax.experimental import pallas as pl
from jax.experimental.pallas import tpu as pltpu
from jax.experimental.pallas import tpu_sc as plsc

N_TYPES = 100
BATCH = 1024
TOTAL = N_TYPES * BATCH
E = TOTAL * 16
EMB = 16
HID = 32
NCLS = 10
NC, NS, L = 2, 16, 16
NW = NC * NS
ER = E // 128
F32 = jnp.float32
I32 = jnp.int32

_MESH = functools.partial(
    plsc.VectorSubcoreMesh, core_axis_name="c", subcore_axis_name="s",
    num_cores=NC, num_subcores=NS)
_SC_PARAMS = pltpu.CompilerParams(use_tc_tiling_on_sc=False)


def _hds(sds):
    return jax.ShapeDtypeStruct(sds[0], sds[1])


def _edge_tx_body(e_ref, o_ref):
    i = e_ref[...]
    q0 = (i.astype(F32) * 0.01).astype(I32)
    r0 = i - q0 * 100
    ge = (r0 >= 100).astype(I32)
    lt = (r0 < 0).astype(I32)
    q = q0 + ge - lt
    r = r0 - 100 * ge + 100 * lt
    o_ref[...] = r * BATCH + q


def _edge_transform(edge_index):
    BE = 65536
    return pl.pallas_call(
        _edge_tx_body,
        grid=(E // BE,),
        in_specs=[pl.BlockSpec((2, BE), lambda g: (0, g))],
        out_specs=pl.BlockSpec((2, BE), lambda g: (0, g)),
        out_shape=jax.ShapeDtypeStruct((2, E), I32),
    )(edge_index)


def _deg_emb_body(dst2, xt, embt, degp, h0, dacc, zb, ones128, dbuf, xbuf,
                  fidx, rows, gsem):
    c = lax.axis_index("c")
    s = lax.axis_index("s")
    wid = s * NC + c

    def zz(i, _):
        zb[pl.ds(i * 16, 16)] = jnp.zeros((16,), F32)
        return _
    lax.fori_loop(0, 50, zz, None)
    for k in range(8):
        pltpu.sync_copy(zb, dacc.at[pl.ds(s * 6400 + k * 800, 800)])

    def oo(i, _):
        ones128[pl.ds(i * 16, 16)] = jnp.ones((16,), F32)
        return _
    lax.fori_loop(0, 8, oo, None)
    plsc.subcore_barrier()

    rows_per_tile = ER // NW
    def dchunk(t, _):
        rb = wid * rows_per_tile + t * 8
        pltpu.sync_copy(dst2.at[pl.ds(rb, 8)], dbuf)
        for j in range(8):
            pltpu.sync_copy(ones128, dacc.at[dbuf.at[j]], add=True)
        return _
    lax.fori_loop(0, rows_per_tile // 8, dchunk, None)

    iota = lax.iota(I32, 16)
    nb0 = wid * (TOTAL // NW)
    def echunk(ch, _):
        nb = nb0 + ch * 640
        pltpu.sync_copy(xt.at[pl.ds(nb, 640)], xbuf)
        for g in range(40):
            xv = xbuf[pl.ds(g * 16, 16)]
            xv = jnp.minimum(jnp.maximum(xv, 0), NCLS)
            row = nb + g * 16 + iota
            fi = (row >> 10) * (NCLS + 1) + xv
            fidx[g // 8, pl.ds((g % 8) * 16, 16)] = fi
        for rr in range(5):
            pltpu.async_copy(embt.at[fidx.at[rr]],
                             rows.at[pl.ds(rr * 128, 128)], gsem).wait()
        pltpu.sync_copy(rows, h0.at[pl.ds(nb, 640)])
        return _
    lax.fori_loop(0, 5, echunk, None)

    plsc.subcore_barrier()
    off = c * TOTAL + s * 6400
    pltpu.sync_copy(dacc.at[pl.ds(s * 6400, 6400)], degp.at[pl.ds(off, 6400)])


def _deg_emb(dst2, xt, embt):
    return pl.kernel(
        _deg_emb_body,
        out_type=(jax.ShapeDtypeStruct((NC * TOTAL,), F32),
                  jax.ShapeDtypeStruct((TOTAL, EMB), F32)),
        mesh=_MESH(),
        compiler_params=_SC_PARAMS,
        scratch_types=(
            pltpu.VMEM_SHARED((TOTAL,), F32),
            pltpu.VMEM((800,), F32),
            pltpu.VMEM((128,), F32),
            pltpu.VMEM((8, 128), I32),
            pltpu.VMEM((640,), I32),
            pltpu.VMEM((5, 128), I32),
            pltpu.VMEM((640, EMB), F32),
            pltpu.SemaphoreType.DMA,
        ),
    )(dst2, xt, embt)


def _agg_body(per_sc_all_edges, src2, dst2, ta, tb, parts, sacc, zb, sidx,
              didx, gb0, gb1, gb2, gb3, gsem0, gsem1, gsem2, gsem3,
              ssem0, ssem1, ssem2, ssem3, isem0, isem1, isem2, isem3):
    c = lax.axis_index("c")
    s = lax.axis_index("s")
    wid = s * NC + c
    isems = (isem0, isem1, isem2, isem3)
    gbs = (gb0, gb1, gb2, gb3)
    gsems = (gsem0, gsem1, gsem2, gsem3)
    ssems = (ssem0, ssem1, ssem2, ssem3)

    def zz(i, _):
        zb[i] = jnp.zeros((16,), F32)
        return _
    lax.fori_loop(0, 100, zz, None)

    def zc(k, _):
        pltpu.sync_copy(zb, sacc.at[pl.ds(s * 6400 + k * 100, 100)])
        return _
    lax.fori_loop(0, 64, zc, None)
    plsc.subcore_barrier()

    def edge_loop(table, row_base, nrows):
        J = 2
        nb = nrows // J

        def load_idx(b, sl):
            rb = row_base + b * J
            pltpu.async_copy(src2.at[pl.ds(rb, J)], sidx.at[sl], isems[sl])
            pltpu.async_copy(dst2.at[pl.ds(rb, J)], didx.at[sl], isems[sl])

        def wait_idx(b, sl):
            rb = row_base + b * J
            pltpu.make_async_copy(src2.at[pl.ds(rb, J)], sidx.at[sl],
                                  isems[sl]).wait()
            pltpu.make_async_copy(dst2.at[pl.ds(rb, J)], didx.at[sl],
                                  isems[sl]).wait()

        def issue_gathers(sl):
            for j in range(J):
                pltpu.async_copy(table.at[sidx.at[sl, j]],
                                 gbs[sl].at[pl.ds(j * 128, 128)], gsems[sl])

        def wait_gathers(sl):
            for j in range(J):
                pltpu.make_async_copy(table.at[sidx.at[sl, j]],
                                      gbs[sl].at[pl.ds(j * 128, 128)],
                                      gsems[sl]).wait()

        def issue_scatters(sl):
            for j in range(J):
                pltpu.async_copy(gbs[sl].at[pl.ds(j * 128, 128)],
                                 sacc.at[didx.at[sl, j]], ssems[sl], add=True)

        def wait_scatters(sl):
            for j in range(J):
                pltpu.make_async_copy(gbs[sl].at[pl.ds(j * 128, 128)],
                                      sacc.at[didx.at[sl, j]], ssems[sl]).wait()

        def process(b, k, static):
            wait_gathers(k)
            issue_scatters(k)
            s1, s2 = (k + 1) % 4, (k + 2) % 4
            def stage_next():
                wait_idx(b + 1, s1)
                issue_gathers(s1)
            def stage_idx():
                wait_scatters(s2)
                load_idx(b + 2, s2)
            if static:
                if b + 1 < nb:
                    stage_next()
                if b + 2 < nb:
                    if b >= 2:
                        stage_idx()
                    else:
                        load_idx(b + 2, s2)
            else:
                pl.when(b + 1 < nb)(stage_next)
                pl.when(b + 2 < nb)(stage_idx)

        load_idx(0, 0)
        load_idx(1, 1)
        wait_idx(0, 0)
        issue_gathers(0)
        for k in range(4):
            process(k, k, True)

        def quad(u, _):
            for k in range(4):
                process(4 * u + k, k, False)
            return _
        lax.fori_loop(1, nb // 4, quad, None)
        for b in range(4 * (nb // 4), nb):
            process(b, b % 4, True)
        for k in range(4):
            wait_scatters((nb - 4 + k) % 4)

    if per_sc_all_edges:
        @pl.when(c == 0)
        def _():
            edge_loop(ta, s * (ER // NS), ER // NS)
        @pl.when(c == 1)
        def _():
            edge_loop(tb, s * (ER // NS), ER // NS)
    else:
        edge_loop(ta, wid * (ER // NW), ER // NW)

    plsc.subcore_barrier()
    off = c * TOTAL + s * 6400
    pltpu.sync_copy(sacc.at[pl.ds(s * 6400, 6400)],
                    parts.at[pl.ds(off, 6400)])


def _agg(src2, dst2, ta, tb, per_sc_all_edges):
    return pl.kernel(
        functools.partial(_agg_body, per_sc_all_edges),
        out_type=jax.ShapeDtypeStruct((NC * TOTAL, EMB), F32),
        mesh=_MESH(),
        compiler_params=_SC_PARAMS,
        scratch_types=(
            pltpu.VMEM_SHARED((TOTAL, EMB), F32),
            pltpu.VMEM((100, EMB), F32),
            pltpu.VMEM((4, 2, 128), I32),
            pltpu.VMEM((4, 2, 128), I32),
            pltpu.VMEM((256, EMB), F32),
            pltpu.VMEM((256, EMB), F32),
            pltpu.VMEM((256, EMB), F32),
            pltpu.VMEM((256, EMB), F32),
        ) + (pltpu.SemaphoreType.DMA,) * 12,
    )(src2, dst2, ta, tb)


def _dinv_body(degp_ref, h0_ref, dinv_ref, g0_ref):
    deg = degp_ref[0] + degp_ref[1] + 1.0
    dinv = lax.rsqrt(deg)[:, None]
    dinv_ref[...] = dinv
    g0_ref[...] = dinv * h0_ref[...]


def _dinv_g0(degp, h0):
    BK = 4096
    return pl.pallas_call(
        _dinv_body,
        grid=(TOTAL // BK,),
        in_specs=[pl.BlockSpec((2, BK), lambda g: (0, g)),
                  pl.BlockSpec((BK, EMB), lambda g: (g, 0))],
        out_specs=[pl.BlockSpec((BK, 1), lambda g: (g, 0)),
                   pl.BlockSpec((BK, EMB), lambda g: (g, 0))],
        out_shape=(jax.ShapeDtypeStruct((TOTAL, 1), F32),
                   jax.ShapeDtypeStruct((TOTAL, EMB), F32)),
    )(degp.reshape(2, TOTAL), h0)


def _layer1_body(parts_ref, g0_ref, dinv_ref, w1t_ref, b1_ref,
                 g1a_ref, g1b_ref):
    ssum = parts_ref[0] + parts_ref[1] + g0_ref[...]
    agg = dinv_ref[...] * ssum
    h1 = jnp.dot(agg, w1t_ref[...], precision=lax.Precision.HIGHEST,
                 preferred_element_type=F32) + b1_ref[...]
    h1 = jnp.maximum(h1, 0.0)
    g1 = dinv_ref[...] * h1
    g1a_ref[...] = g1[:, :EMB]
    g1b_ref[...] = g1[:, EMB:]


def _layer1(parts, g0, dinv, w1t, b1):
    BK = 4096
    return pl.pallas_call(
        _layer1_body,
        grid=(TOTAL // BK,),
        in_specs=[pl.BlockSpec((2, BK, EMB), lambda g: (0, g, 0)),
                  pl.BlockSpec((BK, EMB), lambda g: (g, 0)),
                  pl.BlockSpec((BK, 1), lambda g: (g, 0)),
                  pl.BlockSpec((EMB, HID), lambda g: (0, 0)),
                  pl.BlockSpec((1, HID), lambda g: (0, 0))],
        out_specs=[pl.BlockSpec((BK, EMB), lambda g: (g, 0)),
                   pl.BlockSpec((BK, EMB), lambda g: (g, 0))],
        out_shape=(jax.ShapeDtypeStruct((TOTAL, EMB), F32),
                   jax.ShapeDtypeStruct((TOTAL, EMB), F32)),
    )(parts.reshape(2, TOTAL, EMB), g0, dinv, w1t, b1)


_TPB = 4


def _layer2_body(parts_ref, g1a_ref, g1b_ref, dinv_ref, w2t_ref, b2_ref,
                 hwt_ref, hb_ref, out_ref):
    dinv = dinv_ref[...]
    h0c = dinv * (parts_ref[0] + g1a_ref[...])
    h1c = dinv * (parts_ref[1] + g1b_ref[...])
    h2 = (jnp.dot(h0c, w2t_ref[0], precision=lax.Precision.HIGHEST,
                  preferred_element_type=F32)
          + jnp.dot(h1c, w2t_ref[1], precision=lax.Precision.HIGHEST,
                    preferred_element_type=F32)
          + b2_ref[...])
    for q in range(_TPB):
        lg = jnp.dot(h2[q * BATCH:(q + 1) * BATCH], hwt_ref[q],
                     precision=lax.Precision.HIGHEST,
                     preferred_element_type=F32) + hb_ref[q]
        out_ref[q] = lg


def _layer2_head(parts, g1a, g1b, dinv, w2t, b2, hwt, hb):
    BK = _TPB * BATCH
    return pl.pallas_call(
        _layer2_body,
        grid=(TOTAL // BK,),
        in_specs=[pl.BlockSpec((2, BK, EMB), lambda n: (0, n, 0)),
                  pl.BlockSpec((BK, EMB), lambda n: (n, 0)),
                  pl.BlockSpec((BK, EMB), lambda n: (n, 0)),
                  pl.BlockSpec((BK, 1), lambda n: (n, 0)),
                  pl.BlockSpec((2, EMB, HID), lambda n: (0, 0, 0)),
                  pl.BlockSpec((1, HID), lambda n: (0, 0)),
                  pl.BlockSpec((_TPB, HID, NCLS), lambda n: (n, 0, 0)),
                  pl.BlockSpec((_TPB, 1, NCLS), lambda n: (n, 0, 0))],
        out_specs=pl.BlockSpec((_TPB, BATCH, NCLS), lambda n: (n, 0, 0)),
        out_shape=jax.ShapeDtypeStruct((N_TYPES, BATCH, NCLS), F32),
    )(parts.reshape(2, TOTAL, EMB), g1a, g1b, dinv, w2t.reshape(2, EMB, HID),
      b2, hwt, hb.reshape(N_TYPES, 1, NCLS))


def kernel(x, edge_index, emb_tables, conv1_W, conv1_b, conv2_W, conv2_b,
           head_W, head_b):
    xt = x.reshape(BATCH, N_TYPES).T.reshape(TOTAL)
    embt = emb_tables.reshape(N_TYPES * (NCLS + 1), EMB)
    w1t = conv1_W.T
    b1 = conv1_b.reshape(1, HID)
    w2t = conv2_W.T
    b2 = conv2_b.reshape(1, HID)
    hwt = head_W.transpose(0, 2, 1)

    et = _edge_transform(edge_index)
    src2 = et[0].reshape(ER, 128)
    dst2 = et[1].reshape(ER, 128)

    degp, h0 = _deg_emb(dst2, xt, embt)
    dinv, g0 = _dinv_g0(degp, h0)
    parts1 = _agg(src2, dst2, g0, g0, per_sc_all_edges=False)
    g1a, g1b = _layer1(parts1, g0, dinv, w1t, b1)
    parts2 = _agg(src2, dst2, g1a, g1b, per_sc_all_edges=True)
    lg = _layer2_head(parts2, g1a, g1b, dinv, w2t, b2, hwt, head_b)
    return lg.transpose(1, 0, 2)


# speedup vs baseline: 64.3702x; 1.5512x over previous
import functools

import jax
import jax.numpy as jnp
from jax import lax
from jax.experimental import pallas as pl
from jax.experimental.pallas import tpu as pltpu
from jax.experimental.pallas import tpu_sc as plsc

N_TYPES = 100
BATCH = 1024
TOTAL = N_TYPES * BATCH
E = TOTAL * 16
EMB = 16
HID = 32
NCLS = 10
NC, NS, L = 2, 16, 16
NW = NC * NS
ER = E // 128
F32 = jnp.float32
I32 = jnp.int32

_MESH = functools.partial(
    plsc.VectorSubcoreMesh, core_axis_name="c", subcore_axis_name="s",
    num_cores=NC, num_subcores=NS)
_SC_PARAMS = pltpu.CompilerParams(use_tc_tiling_on_sc=False)


def _hds(sds):
    return jax.ShapeDtypeStruct(sds[0], sds[1])


def _edge_tx_body(e_ref, o_ref):
    i = e_ref[...]
    q0 = (i.astype(F32) * 0.01).astype(I32)
    r0 = i - q0 * 100
    ge = (r0 >= 100).astype(I32)
    lt = (r0 < 0).astype(I32)
    q = q0 + ge - lt
    r = r0 - 100 * ge + 100 * lt
    o_ref[...] = r * BATCH + q


def _edge_transform(edge_index):
    BE = 65536
    return pl.pallas_call(
        _edge_tx_body,
        grid=(E // BE,),
        in_specs=[pl.BlockSpec((2, BE), lambda g: (0, g))],
        out_specs=pl.BlockSpec((2, BE), lambda g: (0, g)),
        out_shape=jax.ShapeDtypeStruct((2, E), I32),
    )(edge_index)


def _deg_emb_body(dst2, xt, embt, degp, h0, dacc, zb, ones128, dbuf, xbuf,
                  fidx, rows, gsem):
    c = lax.axis_index("c")
    s = lax.axis_index("s")
    wid = s * NC + c

    def zz(i, _):
        zb[pl.ds(i * 16, 16)] = jnp.zeros((16,), F32)
        return _
    lax.fori_loop(0, 50, zz, None)
    for k in range(8):
        pltpu.sync_copy(zb, dacc.at[pl.ds(s * 6400 + k * 800, 800)])

    def oo(i, _):
        ones128[pl.ds(i * 16, 16)] = jnp.ones((16,), F32)
        return _
    lax.fori_loop(0, 8, oo, None)
    plsc.subcore_barrier()

    rows_per_tile = ER // NW
    def dchunk(t, _):
        rb = wid * rows_per_tile + t * 8
        pltpu.sync_copy(dst2.at[pl.ds(rb, 8)], dbuf)
        for j in range(8):
            pltpu.sync_copy(ones128, dacc.at[dbuf.at[j]], add=True)
        return _
    lax.fori_loop(0, rows_per_tile // 8, dchunk, None)

    iota = lax.iota(I32, 16)
    nb0 = wid * (TOTAL // NW)
    def echunk(ch, _):
        nb = nb0 + ch * 640
        pltpu.sync_copy(xt.at[pl.ds(nb, 640)], xbuf)
        for g in range(40):
            xv = xbuf[pl.ds(g * 16, 16)]
            xv = jnp.minimum(jnp.maximum(xv, 0), NCLS)
            row = nb + g * 16 + iota
            fi = (row >> 10) * (NCLS + 1) + xv
            fidx[g // 8, pl.ds((g % 8) * 16, 16)] = fi
        for rr in range(5):
            pltpu.async_copy(embt.at[fidx.at[rr]],
                             rows.at[pl.ds(rr * 128, 128)], gsem).wait()
        pltpu.sync_copy(rows, h0.at[pl.ds(nb, 640)])
        return _
    lax.fori_loop(0, 5, echunk, None)

    plsc.subcore_barrier()
    off = c * TOTAL + s * 6400
    pltpu.sync_copy(dacc.at[pl.ds(s * 6400, 6400)], degp.at[pl.ds(off, 6400)])


def _deg_emb(dst2, xt, embt):
    return pl.kernel(
        _deg_emb_body,
        out_type=(jax.ShapeDtypeStruct((NC * TOTAL,), F32),
                  jax.ShapeDtypeStruct((TOTAL, EMB), F32)),
        mesh=_MESH(),
        compiler_params=_SC_PARAMS,
        scratch_types=(
            pltpu.VMEM_SHARED((TOTAL,), F32),
            pltpu.VMEM((800,), F32),
            pltpu.VMEM((128,), F32),
            pltpu.VMEM((8, 128), I32),
            pltpu.VMEM((640,), I32),
            pltpu.VMEM((5, 128), I32),
            pltpu.VMEM((640, EMB), F32),
            pltpu.SemaphoreType.DMA,
        ),
    )(dst2, xt, embt)


def _agg_body(per_sc_all_edges, src2, dst2, ta, tb, parts, sacc, zb, sidx,
              didx, gb0, gb1, gb2, gb3, gsem0, gsem1, gsem2, gsem3,
              ssem0, ssem1, ssem2, ssem3, isem0, isem1, isem2, isem3):
    c = lax.axis_index("c")
    s = lax.axis_index("s")
    wid = s * NC + c
    isems = (isem0, isem1, isem2, isem3)
    gbs = (gb0, gb1, gb2, gb3)
    gsems = (gsem0, gsem1, gsem2, gsem3)
    ssems = (ssem0, ssem1, ssem2, ssem3)

    def zz(i, _):
        zb[i] = jnp.zeros((16,), F32)
        return _
    lax.fori_loop(0, 100, zz, None)

    def zc(k, _):
        pltpu.sync_copy(zb, sacc.at[pl.ds(s * 6400 + k * 100, 100)])
        return _
    lax.fori_loop(0, 64, zc, None)
    plsc.subcore_barrier()

    def edge_loop(table, row_base, nrows):
        J = 2
        nb = nrows // J

        def load_idx(b, sl):
            rb = row_base + b * J
            pltpu.async_copy(src2.at[pl.ds(rb, J)], sidx.at[sl], isems[sl])
            pltpu.async_copy(dst2.at[pl.ds(rb, J)], didx.at[sl], isems[sl])

        def wait_idx(b, sl):
            rb = row_base + b * J
            pltpu.make_async_copy(src2.at[pl.ds(rb, J)], sidx.at[sl],
                                  isems[sl]).wait()
            pltpu.make_async_copy(dst2.at[pl.ds(rb, J)], didx.at[sl],
                                  isems[sl]).wait()

        def issue_gathers(sl):
            for j in range(J):
                pltpu.async_copy(table.at[sidx.at[sl, j]],
                                 gbs[sl].at[pl.ds(j * 128, 128)], gsems[sl])

        def wait_gathers(sl):
            for j in range(J):
                pltpu.make_async_copy(table.at[sidx.at[sl, j]],
                                      gbs[sl].at[pl.ds(j * 128, 128)],
                                      gsems[sl]).wait()

        def issue_scatters(sl):
            for j in range(J):
                pltpu.async_copy(gbs[sl].at[pl.ds(j * 128, 128)],
                                 sacc.at[didx.at[sl, j]], ssems[sl], add=True)

        def wait_scatters(sl):
            for j in range(J):
                pltpu.make_async_copy(gbs[sl].at[pl.ds(j * 128, 128)],
                                      sacc.at[didx.at[sl, j]], ssems[sl]).wait()

        def process(b, k, static):
            wait_gathers(k)
            issue_scatters(k)
            s1, s2 = (k + 1) % 4, (k + 2) % 4
            def stage_next():
                wait_idx(b + 1, s1)
                issue_gathers(s1)
            def stage_idx():
                wait_scatters(s2)
                load_idx(b + 2, s2)
            if static:
                if b + 1 < nb:
                    stage_next()
                if b + 2 < nb:
                    if b >= 2:
                        stage_idx()
                    else:
                        load_idx(b + 2, s2)
            else:
                pl.when(b + 1 < nb)(stage_next)
                pl.when(b + 2 < nb)(stage_idx)

        load_idx(0, 0)
        load_idx(1, 1)
        wait_idx(0, 0)
        issue_gathers(0)
        for k in range(4):
            process(k, k, True)

        def quad(u, _):
            for k in range(4):
                process(4 * u + k, k, False)
            return _
        lax.fori_loop(1, nb // 4, quad, None)
        for b in range(4 * (nb // 4), nb):
            process(b, b % 4, True)
        for k in range(4):
            wait_scatters((nb - 4 + k) % 4)

    if per_sc_all_edges:
        @pl.when(c == 0)
        def _():
            edge_loop(ta, s * (ER // NS), ER // NS)
        @pl.when(c == 1)
        def _():
            edge_loop(tb, s * (ER // NS), ER // NS)
    else:
        edge_loop(ta, wid * (ER // NW), ER // NW)

    plsc.subcore_barrier()
    off = c * TOTAL + s * 6400
    pltpu.sync_copy(sacc.at[pl.ds(s * 6400, 6400)],
                    parts.at[pl.ds(off, 6400)])


def _agg(src2, dst2, ta, tb, per_sc_all_edges):
    return pl.kernel(
        functools.partial(_agg_body, per_sc_all_edges),
        out_type=jax.ShapeDtypeStruct((NC * TOTAL, EMB), F32),
        mesh=_MESH(),
        compiler_params=_SC_PARAMS,
        scratch_types=(
            pltpu.VMEM_SHARED((TOTAL, EMB), F32),
            pltpu.VMEM((100, EMB), F32),
            pltpu.VMEM((4, 2, 128), I32),
            pltpu.VMEM((4, 2, 128), I32),
            pltpu.VMEM((256, EMB), F32),
            pltpu.VMEM((256, EMB), F32),
            pltpu.VMEM((256, EMB), F32),
            pltpu.VMEM((256, EMB), F32),
        ) + (pltpu.SemaphoreType.DMA,) * 12,
    )(src2, dst2, ta, tb)


RP = TOTAL // 8
RS = TOTAL // 128


def _dinv_body(degp_ref, h0_ref, e_ref, dinv_ref, g0_ref):
    deg = degp_ref[0] + degp_ref[1] + 1.0
    dv = lax.rsqrt(deg)
    drep = jnp.dot(dv, e_ref[...], precision=lax.Precision.HIGHEST,
                   preferred_element_type=F32)
    dinv_ref[...] = drep
    g0_ref[...] = drep * h0_ref[...]


def _dinv_g0(degp, h0v, erep):
    BK = 32
    return pl.pallas_call(
        _dinv_body,
        grid=(RS // BK,),
        in_specs=[pl.BlockSpec((2, BK, 128), lambda g: (0, g, 0)),
                  pl.BlockSpec((BK, 2048), lambda g: (g, 0)),
                  pl.BlockSpec((128, 2048), lambda g: (0, 0))],
        out_specs=[pl.BlockSpec((BK, 2048), lambda g: (g, 0)),
                   pl.BlockSpec((BK, 2048), lambda g: (g, 0))],
        out_shape=(jax.ShapeDtypeStruct((RS, 2048), F32),
                   jax.ShapeDtypeStruct((RS, 2048), F32)),
    )(degp.reshape(2, RS, 128), h0v, erep)


def _layer1_body(parts_ref, g0_ref, dinv_ref, w1a_ref, w1b_ref, b1a_ref,
                 b1b_ref, g1a_ref, g1b_ref):
    dinv = dinv_ref[...]
    x = dinv * (parts_ref[0] + parts_ref[1] + g0_ref[...])
    h1a = jnp.dot(x, w1a_ref[...], precision=lax.Precision.HIGHEST,
                  preferred_element_type=F32) + b1a_ref[...]
    h1b = jnp.dot(x, w1b_ref[...], precision=lax.Precision.HIGHEST,
                  preferred_element_type=F32) + b1b_ref[...]
    g1a_ref[...] = dinv * jnp.maximum(h1a, 0.0)
    g1b_ref[...] = dinv * jnp.maximum(h1b, 0.0)


def _layer1(parts, g0v, dinvv, w1a, w1b, b1a, b1b):
    BK = 512
    return pl.pallas_call(
        _layer1_body,
        grid=(RP // BK,),
        in_specs=[pl.BlockSpec((2, BK, 128), lambda g: (0, g, 0)),
                  pl.BlockSpec((BK, 128), lambda g: (g, 0)),
                  pl.BlockSpec((BK, 128), lambda g: (g, 0)),
                  pl.BlockSpec((128, 128), lambda g: (0, 0)),
                  pl.BlockSpec((128, 128), lambda g: (0, 0)),
                  pl.BlockSpec((1, 128), lambda g: (0, 0)),
                  pl.BlockSpec((1, 128), lambda g: (0, 0))],
        out_specs=[pl.BlockSpec((BK, 128), lambda g: (g, 0)),
                   pl.BlockSpec((BK, 128), lambda g: (g, 0))],
        out_shape=(jax.ShapeDtypeStruct((RP, 128), F32),
                   jax.ShapeDtypeStruct((RP, 128), F32)),
    )(parts.reshape(2, RP, 128), g0v, dinvv, w1a, w1b, b1a, b1b)


_TPB = 4


def _layer2_body(parts_ref, g1a_ref, g1b_ref, dinv_ref, w2a_ref, w2b_ref,
                 b2_ref, hw_ref, hb_ref, out_ref):
    dinv = dinv_ref[...]
    h0c = dinv * (parts_ref[0] + g1a_ref[...])
    h1c = dinv * (parts_ref[1] + g1b_ref[...])
    h2 = (jnp.dot(h0c, w2a_ref[...], precision=lax.Precision.HIGHEST,
                  preferred_element_type=F32)
          + jnp.dot(h1c, w2b_ref[...], precision=lax.Precision.HIGHEST,
                    preferred_element_type=F32)
          + b2_ref[...])
    for q in range(_TPB):
        lg = jnp.dot(h2[q * 128:(q + 1) * 128], hw_ref[q],
                     precision=lax.Precision.HIGHEST,
                     preferred_element_type=F32) + hb_ref[q]
        out_ref[q] = lg


def _layer2_head(parts, g1av, g1bv, dinvv, w2a, w2b, b2big, hwbig, hbbig):
    BK = _TPB * 128
    return pl.pallas_call(
        _layer2_body,
        grid=(N_TYPES // _TPB,),
        in_specs=[pl.BlockSpec((2, BK, 128), lambda n: (0, n, 0)),
                  pl.BlockSpec((BK, 128), lambda n: (n, 0)),
                  pl.BlockSpec((BK, 128), lambda n: (n, 0)),
                  pl.BlockSpec((BK, 128), lambda n: (n, 0)),
                  pl.BlockSpec((128, 256), lambda n: (0, 0)),
                  pl.BlockSpec((128, 256), lambda n: (0, 0)),
                  pl.BlockSpec((1, 256), lambda n: (0, 0)),
                  pl.BlockSpec((_TPB, 256, 80), lambda n: (n, 0, 0)),
                  pl.BlockSpec((_TPB, 1, 80), lambda n: (n, 0, 0))],
        out_specs=pl.BlockSpec((_TPB, 128, 80), lambda n: (n, 0, 0)),
        out_shape=jax.ShapeDtypeStruct((N_TYPES, 128, 80), F32),
    )(parts.reshape(2, RP, 128), g1av, g1bv, dinvv, w2a, w2b, b2big,
      hwbig, hbbig)


def kernel(x, edge_index, emb_tables, conv1_W, conv1_b, conv2_W, conv2_b,
           head_W, head_b):
    xt = x.reshape(BATCH, N_TYPES).T.reshape(TOTAL)
    embt = emb_tables.reshape(N_TYPES * (NCLS + 1), EMB)
    eye8 = jnp.eye(8, dtype=F32)
    erep = jnp.kron(jnp.eye(128, dtype=F32), jnp.ones((1, EMB), F32))
    w1t = conv1_W.T
    w1a = jnp.kron(eye8, w1t[:, :EMB])
    w1b = jnp.kron(eye8, w1t[:, EMB:])
    b1a = jnp.tile(conv1_b[:EMB], 8).reshape(1, 128)
    b1b = jnp.tile(conv1_b[EMB:], 8).reshape(1, 128)
    w2t = conv2_W.T
    w2a = jnp.kron(eye8, w2t[:EMB, :])
    w2b = jnp.kron(eye8, w2t[EMB:, :])
    b2big = jnp.tile(conv2_b, 8).reshape(1, 256)
    hwt = head_W.transpose(0, 2, 1)
    hwbig = jnp.einsum('pq,nkc->npkqc', eye8, hwt).reshape(N_TYPES, 256, 80)
    hbbig = jnp.tile(head_b, (1, 8)).reshape(N_TYPES, 1, 80)

    et = _edge_transform(edge_index)
    src2 = et[0].reshape(ER, 128)
    dst2 = et[1].reshape(ER, 128)

    degp, h0 = _deg_emb(dst2, xt, embt)
    dinvv, g0v = _dinv_g0(degp, h0.reshape(RS, 2048), erep)
    dinvp = dinvv.reshape(RP, 128)
    g0 = g0v.reshape(TOTAL, EMB)
    parts1 = _agg(src2, dst2, g0, g0, per_sc_all_edges=False)
    g1av, g1bv = _layer1(parts1, g0v.reshape(RP, 128), dinvp,
                         w1a, w1b, b1a, b1b)
    parts2 = _agg(src2, dst2, g1av.reshape(TOTAL, EMB),
                  g1bv.reshape(TOTAL, EMB), per_sc_all_edges=True)
    lg = _layer2_head(parts2, g1av, g1bv, dinvp, w2a, w2b, b2big,
                      hwbig, hbbig)
    return lg.reshape(N_TYPES, BATCH, NCLS).transpose(1, 0, 2)


# speedup vs baseline: 73.5229x; 1.1422x over previous
import functools

import jax
import jax.numpy as jnp
from jax import lax
from jax.experimental import pallas as pl
from jax.experimental.pallas import tpu as pltpu
from jax.experimental.pallas import tpu_sc as plsc

N_TYPES = 100
BATCH = 1024
TOTAL = N_TYPES * BATCH
E = TOTAL * 16
EMB = 16
HID = 32
NCLS = 10
NC, NS, L = 2, 16, 16
NW = NC * NS
ER = E // 128
F32 = jnp.float32
I32 = jnp.int32

_MESH = functools.partial(
    plsc.VectorSubcoreMesh, core_axis_name="c", subcore_axis_name="s",
    num_cores=NC, num_subcores=NS)
_SC_PARAMS = pltpu.CompilerParams(use_tc_tiling_on_sc=False)


def _hds(sds):
    return jax.ShapeDtypeStruct(sds[0], sds[1])


def _edge_tx_body(e_ref, o_ref):
    i = e_ref[...]
    q0 = (i.astype(F32) * 0.01).astype(I32)
    r0 = i - q0 * 100
    ge = (r0 >= 100).astype(I32)
    lt = (r0 < 0).astype(I32)
    q = q0 + ge - lt
    r = r0 - 100 * ge + 100 * lt
    o_ref[...] = r * BATCH + q


def _edge_transform(edge_index):
    BE = 65536
    return pl.pallas_call(
        _edge_tx_body,
        grid=(E // BE,),
        in_specs=[pl.BlockSpec((2, BE), lambda g: (0, g))],
        out_specs=pl.BlockSpec((2, BE), lambda g: (0, g)),
        out_shape=jax.ShapeDtypeStruct((2, E), I32),
    )(edge_index)


def _deg_emb_body(dst2, xt, embt, degp, h0, dacc, zb, ones128, dbuf, xbuf,
                  fidx, rows, gsem):
    c = lax.axis_index("c")
    s = lax.axis_index("s")
    wid = s * NC + c

    def zz(i, _):
        zb[pl.ds(i * 16, 16)] = jnp.zeros((16,), F32)
        return _
    lax.fori_loop(0, 50, zz, None)
    for k in range(8):
        pltpu.sync_copy(zb, dacc.at[pl.ds(s * 6400 + k * 800, 800)])

    def oo(i, _):
        ones128[pl.ds(i * 16, 16)] = jnp.ones((16,), F32)
        return _
    lax.fori_loop(0, 8, oo, None)
    plsc.subcore_barrier()

    rows_per_tile = ER // NW
    def dchunk(t, _):
        rb = wid * rows_per_tile + t * 8
        pltpu.sync_copy(dst2.at[pl.ds(rb, 8)], dbuf)
        for j in range(8):
            pltpu.sync_copy(ones128, dacc.at[dbuf.at[j]], add=True)
        return _
    lax.fori_loop(0, rows_per_tile // 8, dchunk, None)

    iota = lax.iota(I32, 16)
    nb0 = wid * (TOTAL // NW)
    def echunk(ch, _):
        nb = nb0 + ch * 640
        pltpu.sync_copy(xt.at[pl.ds(nb, 640)], xbuf)
        for g in range(40):
            xv = xbuf[pl.ds(g * 16, 16)]
            xv = jnp.minimum(jnp.maximum(xv, 0), NCLS)
            row = nb + g * 16 + iota
            fi = (row >> 10) * (NCLS + 1) + xv
            fidx[g // 8, pl.ds((g % 8) * 16, 16)] = fi
        for rr in range(5):
            pltpu.async_copy(embt.at[fidx.at[rr]],
                             rows.at[pl.ds(rr * 128, 128)], gsem).wait()
        pltpu.sync_copy(rows, h0.at[pl.ds(nb, 640)])
        return _
    lax.fori_loop(0, 5, echunk, None)

    plsc.subcore_barrier()
    off = c * TOTAL + s * 6400
    pltpu.sync_copy(dacc.at[pl.ds(s * 6400, 6400)], degp.at[pl.ds(off, 6400)])


def _deg_emb(dst2, xt, embt):
    return pl.kernel(
        _deg_emb_body,
        out_type=(jax.ShapeDtypeStruct((NC * TOTAL,), F32),
                  jax.ShapeDtypeStruct((TOTAL, EMB), F32)),
        mesh=_MESH(),
        compiler_params=_SC_PARAMS,
        scratch_types=(
            pltpu.VMEM_SHARED((TOTAL,), F32),
            pltpu.VMEM((800,), F32),
            pltpu.VMEM((128,), F32),
            pltpu.VMEM((8, 128), I32),
            pltpu.VMEM((640,), I32),
            pltpu.VMEM((5, 128), I32),
            pltpu.VMEM((640, EMB), F32),
            pltpu.SemaphoreType.DMA,
        ),
    )(dst2, xt, embt)


def _agg_body(per_sc_all_edges, fw, dt, src2, dst2, ta, tb, parts, sacc, zb, sidx,
              didx, gb0, gb1, gb2, gb3, gsem0, gsem1, gsem2, gsem3,
              ssem0, ssem1, ssem2, ssem3, isem0, isem1, isem2, isem3):
    c = lax.axis_index("c")
    s = lax.axis_index("s")
    wid = s * NC + c
    isems = (isem0, isem1, isem2, isem3)
    gbs = (gb0, gb1, gb2, gb3)
    gsems = (gsem0, gsem1, gsem2, gsem3)
    ssems = (ssem0, ssem1, ssem2, ssem3)

    def zz(i, _):
        zb[i] = jnp.zeros((fw,), dt)
        return _
    lax.fori_loop(0, 100, zz, None)

    def zc(k, _):
        pltpu.sync_copy(zb, sacc.at[pl.ds(s * 6400 + k * 100, 100)])
        return _
    lax.fori_loop(0, 64, zc, None)
    plsc.subcore_barrier()

    def edge_loop(table, row_base, nrows):
        J = 2
        nb = nrows // J

        def load_idx(b, sl):
            rb = row_base + b * J
            pltpu.async_copy(src2.at[pl.ds(rb, J)], sidx.at[sl], isems[sl])
            pltpu.async_copy(dst2.at[pl.ds(rb, J)], didx.at[sl], isems[sl])

        def wait_idx(b, sl):
            rb = row_base + b * J
            pltpu.make_async_copy(src2.at[pl.ds(rb, J)], sidx.at[sl],
                                  isems[sl]).wait()
            pltpu.make_async_copy(dst2.at[pl.ds(rb, J)], didx.at[sl],
                                  isems[sl]).wait()

        def issue_gathers(sl):
            for j in range(J):
                pltpu.async_copy(table.at[sidx.at[sl, j]],
                                 gbs[sl].at[pl.ds(j * 128, 128)], gsems[sl])

        def wait_gathers(sl):
            for j in range(J):
                pltpu.make_async_copy(table.at[sidx.at[sl, j]],
                                      gbs[sl].at[pl.ds(j * 128, 128)],
                                      gsems[sl]).wait()

        def issue_scatters(sl):
            for j in range(J):
                pltpu.async_copy(gbs[sl].at[pl.ds(j * 128, 128)],
                                 sacc.at[didx.at[sl, j]], ssems[sl], add=True)

        def wait_scatters(sl):
            for j in range(J):
                pltpu.make_async_copy(gbs[sl].at[pl.ds(j * 128, 128)],
                                      sacc.at[didx.at[sl, j]], ssems[sl]).wait()

        def process(b, k, static):
            wait_gathers(k)
            issue_scatters(k)
            s1, s2 = (k + 1) % 4, (k + 2) % 4
            def stage_next():
                wait_idx(b + 1, s1)
                issue_gathers(s1)
            def stage_idx():
                wait_scatters(s2)
                load_idx(b + 2, s2)
            if static:
                if b + 1 < nb:
                    stage_next()
                if b + 2 < nb:
                    if b >= 2:
                        stage_idx()
                    else:
                        load_idx(b + 2, s2)
            else:
                pl.when(b + 1 < nb)(stage_next)
                pl.when(b + 2 < nb)(stage_idx)

        load_idx(0, 0)
        load_idx(1, 1)
        wait_idx(0, 0)
        issue_gathers(0)
        for k in range(4):
            process(k, k, True)

        def quad(u, _):
            for k in range(4):
                process(4 * u + k, k, False)
            return _
        lax.fori_loop(1, nb // 4, quad, None)
        for b in range(4 * (nb // 4), nb):
            process(b, b % 4, True)
        for k in range(4):
            wait_scatters((nb - 4 + k) % 4)

    if per_sc_all_edges:
        @pl.when(c == 0)
        def _():
            edge_loop(ta, s * (ER // NS), ER // NS)
        @pl.when(c == 1)
        def _():
            edge_loop(tb, s * (ER // NS), ER // NS)
    else:
        edge_loop(ta, wid * (ER // NW), ER // NW)

    plsc.subcore_barrier()
    off = c * TOTAL + s * 6400
    pltpu.sync_copy(sacc.at[pl.ds(s * 6400, 6400)],
                    parts.at[pl.ds(off, 6400)])


def _agg(src2, dst2, ta, tb, per_sc_all_edges, fw=EMB, dt=F32):
    return pl.kernel(
        functools.partial(_agg_body, per_sc_all_edges, fw, dt),
        out_type=jax.ShapeDtypeStruct((NC * TOTAL, fw), dt),
        mesh=_MESH(),
        compiler_params=_SC_PARAMS,
        scratch_types=(
            pltpu.VMEM_SHARED((TOTAL, fw), dt),
            pltpu.VMEM((100, fw), dt),
            pltpu.VMEM((4, 2, 128), I32),
            pltpu.VMEM((4, 2, 128), I32),
            pltpu.VMEM((256, fw), dt),
            pltpu.VMEM((256, fw), dt),
            pltpu.VMEM((256, fw), dt),
            pltpu.VMEM((256, fw), dt),
        ) + (pltpu.SemaphoreType.DMA,) * 12,
    )(src2, dst2, ta, tb)


RP = TOTAL // 8
RS = TOTAL // 128


def _dinv_body(degp_ref, h0_ref, e_ref, e32_ref, dinv_ref, dinv32_ref, g0_ref):
    deg = degp_ref[0] + degp_ref[1] + 1.0
    dv = lax.rsqrt(deg)
    drep = jnp.dot(dv, e_ref[...], precision=lax.Precision.HIGHEST,
                   preferred_element_type=F32)
    dinv_ref[...] = drep
    dinv32_ref[...] = jnp.dot(dv, e32_ref[...],
                              precision=lax.Precision.HIGHEST,
                              preferred_element_type=F32)
    g0_ref[...] = drep * h0_ref[...]


def _dinv_g0(degp, h0v, erep, erep32):
    BK = 32
    return pl.pallas_call(
        _dinv_body,
        grid=(RS // BK,),
        in_specs=[pl.BlockSpec((2, BK, 128), lambda g: (0, g, 0)),
                  pl.BlockSpec((BK, 2048), lambda g: (g, 0)),
                  pl.BlockSpec((128, 2048), lambda g: (0, 0)),
                  pl.BlockSpec((128, 4096), lambda g: (0, 0))],
        out_specs=[pl.BlockSpec((BK, 2048), lambda g: (g, 0)),
                   pl.BlockSpec((BK, 4096), lambda g: (g, 0)),
                   pl.BlockSpec((BK, 2048), lambda g: (g, 0))],
        out_shape=(jax.ShapeDtypeStruct((RS, 2048), F32),
                   jax.ShapeDtypeStruct((RS, 4096), F32),
                   jax.ShapeDtypeStruct((RS, 2048), F32)),
    )(degp.reshape(2, RS, 128), h0v, erep, erep32)


def _layer1_body(parts_ref, g0_ref, dinv_ref, dinv32_ref, w1_ref, b1_ref,
                 g1_ref):
    dinv = dinv_ref[...]
    x = dinv * (parts_ref[0] + parts_ref[1] + g0_ref[...])
    h1 = jnp.dot(x, w1_ref[...], precision=lax.Precision.HIGHEST,
                 preferred_element_type=F32) + b1_ref[...]
    g1_ref[...] = (dinv32_ref[...] * jnp.maximum(h1, 0.0)).astype(jnp.bfloat16)


def _layer1(parts, g0v, dinvv, dinv32v, w1big, b1big):
    BK = 512
    return pl.pallas_call(
        _layer1_body,
        grid=(RP // BK,),
        in_specs=[pl.BlockSpec((2, BK, 128), lambda g: (0, g, 0)),
                  pl.BlockSpec((BK, 128), lambda g: (g, 0)),
                  pl.BlockSpec((BK, 128), lambda g: (g, 0)),
                  pl.BlockSpec((BK, 256), lambda g: (g, 0)),
                  pl.BlockSpec((128, 256), lambda g: (0, 0)),
                  pl.BlockSpec((1, 256), lambda g: (0, 0))],
        out_specs=pl.BlockSpec((BK, 256), lambda g: (g, 0)),
        out_shape=jax.ShapeDtypeStruct((RP, 256), jnp.bfloat16),
    )(parts.reshape(2, RP, 128), g0v, dinvv, dinv32v, w1big, b1big)


_TPB = 4


def _layer2_body(parts_ref, g1_ref, dinv32_ref, w2_ref, b2_ref, hw_ref,
                 hb_ref, out_ref):
    ssum = (parts_ref[0].astype(F32) + parts_ref[1].astype(F32)
            + g1_ref[...].astype(F32))
    agg = dinv32_ref[...] * ssum
    h2 = jnp.dot(agg, w2_ref[...], precision=lax.Precision.HIGHEST,
                 preferred_element_type=F32) + b2_ref[...]
    for q in range(_TPB):
        lg = jnp.dot(h2[q * 128:(q + 1) * 128], hw_ref[q],
                     precision=lax.Precision.HIGHEST,
                     preferred_element_type=F32) + hb_ref[q]
        out_ref[q] = lg


def _layer2_head(parts, g1v, dinv32v, w2big, b2big, hwbig, hbbig):
    BK = _TPB * 128
    return pl.pallas_call(
        _layer2_body,
        grid=(N_TYPES // _TPB,),
        in_specs=[pl.BlockSpec((2, BK, 256), lambda n: (0, n, 0)),
                  pl.BlockSpec((BK, 256), lambda n: (n, 0)),
                  pl.BlockSpec((BK, 256), lambda n: (n, 0)),
                  pl.BlockSpec((256, 256), lambda n: (0, 0)),
                  pl.BlockSpec((1, 256), lambda n: (0, 0)),
                  pl.BlockSpec((_TPB, 256, 80), lambda n: (n, 0, 0)),
                  pl.BlockSpec((_TPB, 1, 80), lambda n: (n, 0, 0))],
        out_specs=pl.BlockSpec((_TPB, 128, 80), lambda n: (n, 0, 0)),
        out_shape=jax.ShapeDtypeStruct((N_TYPES, 128, 80), F32),
    )(parts.reshape(2, RP, 256), g1v, dinv32v, w2big, b2big, hwbig, hbbig)


def kernel(x, edge_index, emb_tables, conv1_W, conv1_b, conv2_W, conv2_b,
           head_W, head_b):
    xt = x.reshape(BATCH, N_TYPES).T.reshape(TOTAL)
    embt = emb_tables.reshape(N_TYPES * (NCLS + 1), EMB)
    eye8 = jnp.eye(8, dtype=F32)
    erep = jnp.kron(jnp.eye(128, dtype=F32), jnp.ones((1, EMB), F32))
    erep32 = jnp.kron(jnp.eye(128, dtype=F32), jnp.ones((1, HID), F32))
    w1big = jnp.kron(eye8, conv1_W.T)
    b1big = jnp.tile(conv1_b, 8).reshape(1, 256)
    w2big = jnp.kron(eye8, conv2_W.T)
    b2big = jnp.tile(conv2_b, 8).reshape(1, 256)
    hwt = head_W.transpose(0, 2, 1)
    hwbig = jnp.einsum('pq,nkc->npkqc', eye8, hwt).reshape(N_TYPES, 256, 80)
    hbbig = jnp.tile(head_b, (1, 8)).reshape(N_TYPES, 1, 80)

    et = _edge_transform(edge_index)
    src2 = et[0].reshape(ER, 128)
    dst2 = et[1].reshape(ER, 128)

    degp, h0 = _deg_emb(dst2, xt, embt)
    dinvv, dinv32v, g0v = _dinv_g0(degp, h0.reshape(RS, 2048), erep, erep32)
    dinvp = dinvv.reshape(RP, 128)
    dinv32p = dinv32v.reshape(RP, 256)
    g0 = g0v.reshape(TOTAL, EMB)
    parts1 = _agg(src2, dst2, g0, g0, per_sc_all_edges=False)
    g1v = _layer1(parts1, g0v.reshape(RP, 128), dinvp, dinv32p, w1big, b1big)
    g1 = g1v.reshape(TOTAL, HID)
    parts2 = _agg(src2, dst2, g1, g1, per_sc_all_edges=False,
                  fw=HID, dt=jnp.bfloat16)
    lg = _layer2_head(parts2, g1v, dinv32p, w2big, b2big, hwbig, hbbig)
    return lg.reshape(N_TYPES, BATCH, NCLS).transpose(1, 0, 2)
